# baseline (speedup 1.0000x reference)
import functools

import jax
import jax.numpy as jnp
import numpy as np
from jax import lax
from jax.experimental import pallas as pl
from jax.experimental.pallas import tpu as pltpu
from jax.experimental.pallas import tpu_sc as plsc

NC, NS, LANES = 2, 16, 16
NW = NC * NS
NNODES = 10000
NEDGES = 320000
CH = 128
NCHUNK = 79
EPAD = NW * NCHUNK * CH
NGRAPH = 100
KTOP = 30
NPAD = 10240
ZSL = NPAD // NS
NEGZ = -1e30
NEG = -3.0
BIG = 1 << 30


def _sc_mesh():
    return plsc.VectorSubcoreMesh(core_axis_name="c", subcore_axis_name="s",
                                  num_cores=NC, num_subcores=NS)


def _deg_body(dst_hbm, zr_hbm, out_hbm, dst_v, ones_v, acc_sh, dsem):
    cid = lax.axis_index("c")
    sid = lax.axis_index("s")
    wid = cid * NS + sid
    pltpu.sync_copy(dst_hbm.at[pl.ds(wid * NCHUNK, NCHUNK)], dst_v)
    for i in range(8):
        ones_v[pl.ds(i * LANES, LANES)] = jnp.full((LANES,), 1.0, jnp.float32)

    @pl.when(sid == 0)
    def _():
        pltpu.sync_copy(zr_hbm, acc_sh)
    plsc.subcore_barrier()

    def chunk(ch, carry):
        pltpu.async_copy(ones_v.at[pl.ds(0, CH)], acc_sh.at[dst_v.at[ch]],
                         dsem, add=True)
        return carry
    lax.fori_loop(0, NCHUNK, chunk, 0)

    def drain(ch, carry):
        pltpu.make_async_copy(ones_v.at[pl.ds(0, CH)],
                              acc_sh.at[dst_v.at[ch]], dsem).wait()
        return carry
    lax.fori_loop(0, NCHUNK, drain, 0)
    plsc.subcore_barrier()

    @pl.when(sid == 0)
    def _():
        pltpu.sync_copy(acc_sh, out_hbm.at[cid])


def _make_deg():
    return pl.kernel(
        _deg_body,
        out_type=jax.ShapeDtypeStruct((NC, NPAD), jnp.float32),
        mesh=_sc_mesh(),
        compiler_params=pltpu.CompilerParams(use_tc_tiling_on_sc=False, needs_layout_passes=False),
        scratch_types=[
            pltpu.VMEM((NCHUNK, CH), jnp.int32),
            pltpu.VMEM((128,), jnp.float32),
            pltpu.VMEM_SHARED((NPAD,), jnp.float32),
            pltpu.SemaphoreType.DMA,
        ],
    )


RING = 8
LOOKAHEAD = 4


def _mp_body(hs_hbm, src_hbm, dst_hbm, zr_hbm, out_hbm,
             src_v, dst_v, rows_v, acc_sh, gsems, ssems):
    cid = lax.axis_index("c")
    sid = lax.axis_index("s")
    wid = cid * NS + sid
    pltpu.sync_copy(src_hbm.at[pl.ds(wid * NCHUNK, NCHUNK)], src_v)
    pltpu.sync_copy(dst_hbm.at[pl.ds(wid * NCHUNK, NCHUNK)], dst_v)

    @pl.when(sid == 0)
    def _():
        pltpu.sync_copy(zr_hbm, acc_sh)
    plsc.subcore_barrier()

    def gdesc(c, j):
        return pltpu.make_async_copy(hs_hbm.at[src_v.at[c]], rows_v.at[j],
                                     gsems.at[j])

    def sdesc(c, j):
        return pltpu.make_async_copy(rows_v.at[j], acc_sh.at[dst_v.at[c]],
                                     ssems.at[j])

    def step(i, carry):
        p0 = i * RING
        for j in range(RING):
            p = p0 + j
            ja = (j - LOOKAHEAD) % RING

            @pl.when((p >= RING) & (p < NCHUNK))
            def _():
                sdesc(p - RING, j).wait()

            @pl.when(p < NCHUNK)
            def _():
                pltpu.async_copy(hs_hbm.at[src_v.at[p]], rows_v.at[j],
                                 gsems.at[j])
            ca = p - LOOKAHEAD

            @pl.when((ca >= 0) & (ca < NCHUNK))
            def _():
                gdesc(ca, ja).wait()
                pltpu.async_copy(rows_v.at[ja], acc_sh.at[dst_v.at[ca]],
                                 ssems.at[ja], add=True)
        return carry
    nsteps = (NCHUNK + LOOKAHEAD + RING - 1) // RING
    lax.fori_loop(0, nsteps, step, 0)
    for j in range(RING):
        c = NCHUNK - RING + j
        sdesc(c, c % RING).wait()
    plsc.subcore_barrier()

    @pl.when(sid == 0)
    def _():
        pltpu.sync_copy(acc_sh, out_hbm.at[cid])


def _make_mp(feat, nnodes=NPAD):
    return pl.kernel(
        _mp_body,
        out_type=jax.ShapeDtypeStruct((NC, nnodes, feat), jnp.float32),
        mesh=_sc_mesh(),
        compiler_params=pltpu.CompilerParams(use_tc_tiling_on_sc=False, needs_layout_passes=False),
        scratch_types=[
            pltpu.VMEM((NCHUNK, CH), jnp.int32),
            pltpu.VMEM((NCHUNK, CH), jnp.int32),
            pltpu.VMEM((RING, CH, feat), jnp.float32),
            pltpu.VMEM_SHARED((nnodes, feat), jnp.float32),
            pltpu.SemaphoreType.DMA((RING,)),
            pltpu.SemaphoreType.DMA((RING,)),
        ],
    )


def _topkz_body(p4_hbm, hs4_hbm, dinv_hbm, b4_hbm, xc_hbm, starts_hbm,
                counts_hbm, out_hbm,
                pa_v, pb_v, hb_v, dv_v, b4s_v, zl_v, z_sh, zv_v,
                starts_v, counts_v, idx_v, rows_v, sem):
    cid = lax.axis_index("c")
    sid = lax.axis_index("s")
    wid = cid * NS + sid
    iota = lax.iota(jnp.int32, LANES)
    lane0 = iota == 0
    base = sid * ZSL

    pltpu.sync_copy(p4_hbm.at[0, pl.ds(base, ZSL)], pa_v)
    pltpu.sync_copy(p4_hbm.at[1, pl.ds(base, ZSL)], pb_v)
    pltpu.sync_copy(hs4_hbm.at[pl.ds(base, ZSL)], hb_v)
    pltpu.sync_copy(dinv_hbm.at[sid], dv_v)
    pltpu.sync_copy(b4_hbm, b4s_v)
    pltpu.sync_copy(starts_hbm, starts_v)
    pltpu.sync_copy(counts_hbm, counts_v)
    b4 = b4s_v[pl.ds(0, LANES)][0]
    zeros16 = jnp.zeros((LANES,), jnp.int32)

    def zchunk(i, carry):
        r16 = i * LANES + iota
        ga = plsc.load_gather(pa_v, [r16, zeros16])
        gb = plsc.load_gather(pb_v, [r16, zeros16])
        gh = plsc.load_gather(hb_v, [r16, zeros16])
        z16 = dv_v[pl.ds(i * LANES, LANES)] * (ga + gb + gh) + b4
        z16 = jnp.where(base + r16 < NNODES, z16, jnp.float32(1e30))
        zl_v[pl.ds(i * LANES, LANES)] = z16
        return carry
    lax.fori_loop(0, ZSL // LANES, zchunk, 0)
    pltpu.sync_copy(zl_v, z_sh.at[pl.ds(base, ZSL)])
    plsc.subcore_barrier()
    pltpu.sync_copy(z_sh, zv_v)

    def minchunk(i, m):
        return jnp.minimum(m, jnp.min(zv_v[pl.ds(i * LANES, LANES)]))
    zmin = lax.fori_loop(0, NNODES // LANES, minchunk, jnp.float32(1e30))
    zminv = jnp.full((LANES,), zmin, jnp.float32)
    fillv = -2.0 / (jnp.exp(2.0 * zminv) + 1.0)

    for rep in range(4):
        g = wid + rep * NW

        @pl.when(g < NGRAPH)
        def _process():
            start = starts_v[pl.ds(g, LANES)][0]
            cnt = counts_v[pl.ds(g, LANES)][0]
            end = start + cnt
            nsel = jnp.minimum(cnt, KTOP)
            c0 = start // LANES
            c1 = (end + LANES - 1) // LANES

            def select(k, carry):
                def scan_chunk(c, mbi):
                    m, bi = mbi
                    v = zv_v[pl.ds(c * LANES, LANES)]
                    p = c * LANES + iota
                    valid = (p >= start) & (p < end)
                    vv = jnp.where(valid, v, NEGZ)
                    cm = jnp.max(vv)
                    ci = jnp.min(jnp.where(vv == cm, p, BIG))
                    better = cm > m
                    return (jnp.where(better, cm, m),
                            jnp.where(better, ci, bi))
                m, bi = lax.fori_loop(c0, c1, scan_chunk,
                                      (jnp.float32(NEGZ), jnp.int32(BIG)))
                biv = jnp.full((LANES,), bi, jnp.int32)
                plsc.store_scatter(idx_v, [jnp.full((LANES,), k, jnp.int32)],
                                   biv, mask=lane0)
                plsc.store_scatter(zv_v, [biv],
                                   jnp.full((LANES,), NEGZ, jnp.float32),
                                   mask=lane0)
                return carry
            lax.fori_loop(0, nsel, select, 0)

            def pad(k, carry):
                plsc.store_scatter(idx_v, [jnp.full((LANES,), k, jnp.int32)],
                                   jnp.zeros((LANES,), jnp.int32), mask=lane0)
                return carry
            lax.fori_loop(nsel, KTOP + 2, pad, 0)

            pltpu.async_copy(xc_hbm.at[idx_v], rows_v, sem).wait()

            def fillrow(k, carry):
                for j in range(6):
                    rows_v[k, pl.ds(j * LANES, LANES)] = fillv
                return carry
            lax.fori_loop(nsel, KTOP, fillrow, 0)
            pltpu.sync_copy(rows_v.at[pl.ds(0, KTOP)], out_hbm.at[g])


def _make_topkz():
    return pl.kernel(
        _topkz_body,
        out_type=jax.ShapeDtypeStruct((NGRAPH, KTOP, 96), jnp.float32),
        mesh=_sc_mesh(),
        compiler_params=pltpu.CompilerParams(use_tc_tiling_on_sc=False, needs_layout_passes=False),
        scratch_types=[
            pltpu.VMEM((ZSL, 16), jnp.float32),
            pltpu.VMEM((ZSL, 16), jnp.float32),
            pltpu.VMEM((ZSL, 16), jnp.float32),
            pltpu.VMEM((ZSL,), jnp.float32),
            pltpu.VMEM((128,), jnp.float32),
            pltpu.VMEM((ZSL,), jnp.float32),
            pltpu.VMEM_SHARED((NPAD,), jnp.float32),
            pltpu.VMEM((NPAD,), jnp.float32),
            pltpu.VMEM((128,), jnp.int32),
            pltpu.VMEM((128,), jnp.int32),
            pltpu.VMEM((KTOP + 2,), jnp.int32),
            pltpu.VMEM((KTOP + 2, 96), jnp.float32),
            pltpu.SemaphoreType.DMA,
        ],
    )


def _l1_body(dp_ref, x_ref, w_ref, batch_ref, o_hs, o_dinv,
             o_counts, o_starts):
    deg = dp_ref[0][:NNODES] + dp_ref[1][:NNODES] + 1.0
    dinv = lax.rsqrt(deg)
    o_dinv[...] = dinv
    o_hs[...] = jnp.dot(x_ref[...], w_ref[...],
                        preferred_element_type=jnp.float32) * dinv
    gids = lax.broadcasted_iota(jnp.int32, (128, NNODES), 0)
    onehot = (batch_ref[...] == gids).astype(jnp.float32)
    counts = jnp.sum(onehot, axis=1, keepdims=True)
    r = lax.broadcasted_iota(jnp.int32, (128, 128), 0)
    c = lax.broadcasted_iota(jnp.int32, (128, 128), 1)
    tril = (c < r).astype(jnp.float32)
    starts = jnp.dot(tril, counts, preferred_element_type=jnp.float32)
    o_counts[...] = counts.astype(jnp.int32)
    o_starts[...] = starts.astype(jnp.int32)


def _lmid_body(p_ref, hs_ref, dinv_ref, b_ref, wn_ref, o_a, o_hsn):
    dinv = dinv_ref[...]
    a = jnp.tanh(dinv * (p_ref[0][:NNODES] + p_ref[1][:NNODES]
                      + hs_ref[...]) + b_ref[...])
    o_a[...] = a
    o_hsn[...] = jnp.dot(a, wn_ref[...],
                         preferred_element_type=jnp.float32) * dinv


def _head_body(xkf_ref, w1e_ref, w1o_ref, b1_ref, w2_ref, b2_ref,
               wl1_ref, bl1_ref, wl2_ref, bl2_ref, o_ref):
    xkf = xkf_ref[...]
    ce = jnp.maximum(jnp.dot(xkf, w1e_ref[...],
                             preferred_element_type=jnp.float32)
                     + b1_ref[...], 0.0)
    co = jnp.maximum(jnp.dot(xkf, w1o_ref[...],
                             preferred_element_type=jnp.float32)
                     + b1_ref[...], 0.0)
    r1 = jnp.maximum(ce, co)
    r2 = jnp.maximum(jnp.dot(r1, w2_ref[...],
                             preferred_element_type=jnp.float32)
                     + b2_ref[...], 0.0)
    p = r2[:, 0:512]
    for k in range(1, 5):
        p = jnp.maximum(p, r2[:, k * 32:k * 32 + 512])
    h = jnp.maximum(jnp.dot(p, wl1_ref[...],
                            preferred_element_type=jnp.float32)
                    + bl1_ref[...], 0.0)
    logits = jnp.dot(h, wl2_ref[...],
                     preferred_element_type=jnp.float32) + bl2_ref[...]
    m = jnp.max(logits, axis=1, keepdims=True)
    shifted = logits - m
    o_ref[...] = shifted - jnp.log(jnp.sum(jnp.exp(shifted), axis=1,
                                           keepdims=True))


_SEL_E = np.zeros((96, 24, 2), np.float32)
_SEL_O = np.zeros((96, 24, 2), np.float32)
for _u in range(24):
    for _k in range(2):
        _SEL_E[4 * _u + _k, _u, _k] = 1.0
        if 4 * _u + 2 + _k < 96:
            _SEL_O[4 * _u + 2 + _k, _u, _k] = 1.0
_SEL2 = np.zeros((24, 20, 5), np.float32)
for _t in range(20):
    for _k in range(5):
        _SEL2[_t + _k, _t, _k] = 1.0


def _tc_call(body, out_shape):
    return pl.pallas_call(body, out_shape=out_shape)


def kernel(x, edge_index, batch, W1, b1, W2, b2, W3, b3, W4, b4,
           Wc1, bc1, Wc2, bc2, Wl1, bl1, Wl2, bl2):
    f32 = jnp.float32
    epad = jnp.concatenate(
        [edge_index,
         jnp.asarray(np.stack([np.zeros(EPAD - NEDGES, np.int32),
                               np.full(EPAD - NEDGES, NNODES, np.int32)]))],
        axis=1)
    src = epad[0].reshape(NW * NCHUNK, CH)
    dst = epad[1].reshape(NW * NCHUNK, CH)

    zr1 = jnp.zeros((NPAD,), f32)
    zr32 = jnp.zeros((NPAD, 32), f32)
    zr16 = jnp.zeros((NPAD, 16), f32)

    degp = _make_deg()(dst, zr1)
    degp3 = degp.reshape(NC, NPAD, 1)

    hs1, dinv, counts2, starts2 = _tc_call(
        _l1_body,
        (jax.ShapeDtypeStruct((NNODES, 32), f32),
         jax.ShapeDtypeStruct((NNODES, 1), f32),
         jax.ShapeDtypeStruct((128, 1), jnp.int32),
         jax.ShapeDtypeStruct((128, 1), jnp.int32)),
    )(degp3, x, W1, batch.reshape(1, NNODES))

    mp32 = _make_mp(32)
    p1 = mp32(hs1, src, dst, zr32)
    a1, hs2 = _tc_call(
        _lmid_body,
        (jax.ShapeDtypeStruct((NNODES, 32), f32),
         jax.ShapeDtypeStruct((NNODES, 32), f32)),
    )(p1, hs1, dinv, b1.reshape(1, 32), W2)

    p2 = mp32(hs2, src, dst, zr32)
    a2, hs3 = _tc_call(
        _lmid_body,
        (jax.ShapeDtypeStruct((NNODES, 32), f32),
         jax.ShapeDtypeStruct((NNODES, 32), f32)),
    )(p2, hs2, dinv, b2.reshape(1, 32), W3)

    p3 = mp32(hs3, src, dst, zr32)
    W4p = jnp.pad(W4, ((0, 0), (0, 15)))
    a3, hs4 = _tc_call(
        _lmid_body,
        (jax.ShapeDtypeStruct((NNODES, 32), f32),
         jax.ShapeDtypeStruct((NNODES, 16), f32)),
    )(p3, hs3, dinv, b3.reshape(1, 32), W4p)

    hs4p = jnp.pad(hs4, ((0, NPAD - NNODES), (0, 0)))
    p4 = _make_mp(16)(hs4p, src, dst, zr16)

    xc = jnp.concatenate([a1, a2, a3], axis=1)
    dinvp = jnp.pad(dinv.reshape(NNODES), (0, NPAD - NNODES)).reshape(NS, ZSL)
    b4p = jnp.zeros((128,), f32).at[0].set(b4[0])
    starts1 = starts2.reshape(128)
    counts1 = counts2.reshape(128)

    xk = _make_topkz()(p4, hs4p, dinvp, b4p, xc, starts1, counts1)

    xkf = xk.reshape(NGRAPH, KTOP * 96)
    W1e = jnp.einsum('oik,huk->ihuo', Wc1,
                     jnp.asarray(_SEL_E)).reshape(KTOP * 96, 384)
    W1o = jnp.einsum('oik,huk->ihuo', Wc1,
                     jnp.asarray(_SEL_O)).reshape(KTOP * 96, 384)
    W2b = jnp.einsum('oik,utk->uito', Wc2,
                     jnp.asarray(_SEL2)).reshape(384, 640)
    Wl1p = Wl1.reshape(32, 16, 128).transpose(1, 0, 2).reshape(512, 128)

    out = _tc_call(
        _head_body, jax.ShapeDtypeStruct((NGRAPH, 10), f32),
    )(xkf, W1e, W1o, jnp.tile(bc1.reshape(1, 16), (1, 24)),
      W2b, jnp.tile(bc2.reshape(1, 32), (1, 20)),
      Wl1p, bl1.reshape(1, 128), Wl2, bl2.reshape(1, 10))
    return out


# speedup vs baseline: 1.4056x; 1.4056x over previous
import functools

import jax
import jax.numpy as jnp
import numpy as np
from jax import lax
from jax.experimental import pallas as pl
from jax.experimental.pallas import tpu as pltpu
from jax.experimental.pallas import tpu_sc as plsc

NC, NS, LANES = 2, 16, 16
NW = NC * NS
NNODES = 10000
NEDGES = 320000
CH = 128
NCHUNK = 79
EPAD = NW * NCHUNK * CH
NGRAPH = 100
KTOP = 30
NPAD = 10240
ZSL = NPAD // NS
NEGZ = -1e30
NEG = -3.0
BIG = 1 << 30


def _sc_mesh():
    return plsc.VectorSubcoreMesh(core_axis_name="c", subcore_axis_name="s",
                                  num_cores=NC, num_subcores=NS)


def _deg_body(dst_hbm, zr_hbm, out_hbm, dst_v, ones_v, acc_sh, dsem):
    cid = lax.axis_index("c")
    sid = lax.axis_index("s")
    wid = cid * NS + sid
    pltpu.sync_copy(dst_hbm.at[pl.ds(wid * NCHUNK, NCHUNK)], dst_v)
    for i in range(8):
        ones_v[pl.ds(i * LANES, LANES)] = jnp.full((LANES,), 1.0, jnp.float32)

    @pl.when(sid == 0)
    def _():
        pltpu.sync_copy(zr_hbm, acc_sh)
    plsc.subcore_barrier()

    def chunk(ch, carry):
        pltpu.async_copy(ones_v.at[pl.ds(0, CH)], acc_sh.at[dst_v.at[ch]],
                         dsem, add=True)
        return carry
    lax.fori_loop(0, NCHUNK, chunk, 0)

    def drain(ch, carry):
        pltpu.make_async_copy(ones_v.at[pl.ds(0, CH)],
                              acc_sh.at[dst_v.at[ch]], dsem).wait()
        return carry
    lax.fori_loop(0, NCHUNK, drain, 0)
    plsc.subcore_barrier()

    @pl.when(sid == 0)
    def _():
        pltpu.sync_copy(acc_sh, out_hbm.at[cid])


def _make_deg():
    return pl.kernel(
        _deg_body,
        out_type=jax.ShapeDtypeStruct((NC, NPAD), jnp.float32),
        mesh=_sc_mesh(),
        compiler_params=pltpu.CompilerParams(use_tc_tiling_on_sc=False, needs_layout_passes=False),
        scratch_types=[
            pltpu.VMEM((NCHUNK, CH), jnp.int32),
            pltpu.VMEM((128,), jnp.float32),
            pltpu.VMEM_SHARED((NPAD,), jnp.float32),
            pltpu.SemaphoreType.DMA,
        ],
    )


RING = 8
LOOKAHEAD = 4


def _mp_body(hs_hbm, src_hbm, dst_hbm, zr_hbm, out_hbm,
             src_v, dst_v, rows_v, acc_sh, gsems, ssems):
    cid = lax.axis_index("c")
    sid = lax.axis_index("s")
    wid = cid * NS + sid
    pltpu.sync_copy(src_hbm.at[pl.ds(wid * NCHUNK, NCHUNK)], src_v)
    pltpu.sync_copy(dst_hbm.at[pl.ds(wid * NCHUNK, NCHUNK)], dst_v)

    @pl.when(sid == 0)
    def _():
        pltpu.sync_copy(zr_hbm, acc_sh)
    plsc.subcore_barrier()

    def gdesc(c, j):
        return pltpu.make_async_copy(hs_hbm.at[src_v.at[c]], rows_v.at[j],
                                     gsems.at[j])

    def sdesc(c, j):
        return pltpu.make_async_copy(rows_v.at[j], acc_sh.at[dst_v.at[c]],
                                     ssems.at[j])

    def step(i, carry):
        p0 = i * RING
        for j in range(RING):
            p = p0 + j
            ja = (j - LOOKAHEAD) % RING

            @pl.when((p >= RING) & (p < NCHUNK))
            def _():
                sdesc(p - RING, j).wait()

            @pl.when(p < NCHUNK)
            def _():
                pltpu.async_copy(hs_hbm.at[src_v.at[p]], rows_v.at[j],
                                 gsems.at[j])
            ca = p - LOOKAHEAD

            @pl.when((ca >= 0) & (ca < NCHUNK))
            def _():
                gdesc(ca, ja).wait()
                pltpu.async_copy(rows_v.at[ja], acc_sh.at[dst_v.at[ca]],
                                 ssems.at[ja], add=True)
        return carry
    nsteps = (NCHUNK + LOOKAHEAD + RING - 1) // RING
    lax.fori_loop(0, nsteps, step, 0)
    for j in range(RING):
        c = NCHUNK - RING + j
        sdesc(c, c % RING).wait()
    plsc.subcore_barrier()

    @pl.when(sid == 0)
    def _():
        pltpu.sync_copy(acc_sh, out_hbm.at[cid])


def _make_mp(feat, nnodes=NPAD):
    return pl.kernel(
        _mp_body,
        out_type=jax.ShapeDtypeStruct((NC, nnodes, feat), jnp.float32),
        mesh=_sc_mesh(),
        compiler_params=pltpu.CompilerParams(use_tc_tiling_on_sc=False, needs_layout_passes=False),
        scratch_types=[
            pltpu.VMEM((NCHUNK, CH), jnp.int32),
            pltpu.VMEM((NCHUNK, CH), jnp.int32),
            pltpu.VMEM((RING, CH, feat), jnp.float32),
            pltpu.VMEM_SHARED((nnodes, feat), jnp.float32),
            pltpu.SemaphoreType.DMA((RING,)),
            pltpu.SemaphoreType.DMA((RING,)),
        ],
    )


def _topkz_body(p4_hbm, hs4_hbm, dinv_hbm, b4_hbm, xc_hbm, starts_hbm,
                counts_hbm, out_hbm,
                pa_v, pb_v, hb_v, dv_v, b4s_v, zl_v, z_sh, zv_v,
                starts_v, counts_v, idx_v, rows_v, sem):
    cid = lax.axis_index("c")
    sid = lax.axis_index("s")
    wid = cid * NS + sid
    iota = lax.iota(jnp.int32, LANES)
    lane0 = iota == 0
    base = sid * ZSL

    pltpu.sync_copy(p4_hbm.at[0, pl.ds(base, ZSL)], pa_v)
    pltpu.sync_copy(p4_hbm.at[1, pl.ds(base, ZSL)], pb_v)
    pltpu.sync_copy(hs4_hbm.at[pl.ds(base, ZSL)], hb_v)
    pltpu.sync_copy(dinv_hbm.at[sid], dv_v)
    pltpu.sync_copy(b4_hbm, b4s_v)
    pltpu.sync_copy(starts_hbm, starts_v)
    pltpu.sync_copy(counts_hbm, counts_v)
    b4 = b4s_v[pl.ds(0, LANES)][0]
    zeros16 = jnp.zeros((LANES,), jnp.int32)

    def zchunk(i, carry):
        r16 = i * LANES + iota
        ga = plsc.load_gather(pa_v, [r16, zeros16])
        gb = plsc.load_gather(pb_v, [r16, zeros16])
        gh = plsc.load_gather(hb_v, [r16, zeros16])
        z16 = dv_v[pl.ds(i * LANES, LANES)] * (ga + gb + gh) + b4
        z16 = jnp.where(base + r16 < NNODES, z16, jnp.float32(1e30))
        zl_v[pl.ds(i * LANES, LANES)] = z16
        return carry
    lax.fori_loop(0, ZSL // LANES, zchunk, 0)
    pltpu.sync_copy(zl_v, z_sh.at[pl.ds(base, ZSL)])
    plsc.subcore_barrier()
    pltpu.sync_copy(z_sh, zv_v)

    def minchunk(i, m):
        return jnp.minimum(m, jnp.min(zv_v[pl.ds(i * LANES, LANES)]))
    zmin = lax.fori_loop(0, NNODES // LANES, minchunk, jnp.float32(1e30))
    zminv = jnp.full((LANES,), zmin, jnp.float32)
    fillv = -2.0 / (jnp.exp(2.0 * zminv) + 1.0)

    for rep in range(4):
        g = wid + rep * NW

        @pl.when(g < NGRAPH)
        def _process():
            start = starts_v[pl.ds(g, LANES)][0]
            cnt = counts_v[pl.ds(g, LANES)][0]
            end = start + cnt
            nsel = jnp.minimum(cnt, KTOP)
            c0 = start // LANES
            c1 = (end + LANES - 1) // LANES

            def select(k, carry):
                def scan_chunk(c, mbi):
                    m, bi = mbi
                    v = zv_v[pl.ds(c * LANES, LANES)]
                    p = c * LANES + iota
                    valid = (p >= start) & (p < end)
                    vv = jnp.where(valid, v, NEGZ)
                    cm = jnp.max(vv)
                    ci = jnp.min(jnp.where(vv == cm, p, BIG))
                    better = cm > m
                    return (jnp.where(better, cm, m),
                            jnp.where(better, ci, bi))
                m, bi = lax.fori_loop(c0, c1, scan_chunk,
                                      (jnp.float32(NEGZ), jnp.int32(BIG)))
                biv = jnp.full((LANES,), bi, jnp.int32)
                plsc.store_scatter(idx_v, [jnp.full((LANES,), k, jnp.int32)],
                                   biv, mask=lane0)
                plsc.store_scatter(zv_v, [biv],
                                   jnp.full((LANES,), NEGZ, jnp.float32),
                                   mask=lane0)
                return carry
            lax.fori_loop(0, nsel, select, 0)

            def pad(k, carry):
                plsc.store_scatter(idx_v, [jnp.full((LANES,), k, jnp.int32)],
                                   jnp.zeros((LANES,), jnp.int32), mask=lane0)
                return carry
            lax.fori_loop(nsel, KTOP + 2, pad, 0)

            pltpu.async_copy(xc_hbm.at[idx_v], rows_v, sem).wait()

            def fillrow(k, carry):
                for j in range(6):
                    rows_v[k, pl.ds(j * LANES, LANES)] = fillv
                return carry
            lax.fori_loop(nsel, KTOP, fillrow, 0)
            pltpu.sync_copy(rows_v.at[pl.ds(0, KTOP)], out_hbm.at[g])


def _make_topkz():
    return pl.kernel(
        _topkz_body,
        out_type=jax.ShapeDtypeStruct((NGRAPH, KTOP, 96), jnp.float32),
        mesh=_sc_mesh(),
        compiler_params=pltpu.CompilerParams(use_tc_tiling_on_sc=False, needs_layout_passes=False),
        scratch_types=[
            pltpu.VMEM((ZSL, 16), jnp.float32),
            pltpu.VMEM((ZSL, 16), jnp.float32),
            pltpu.VMEM((ZSL, 16), jnp.float32),
            pltpu.VMEM((ZSL,), jnp.float32),
            pltpu.VMEM((128,), jnp.float32),
            pltpu.VMEM((ZSL,), jnp.float32),
            pltpu.VMEM_SHARED((NPAD,), jnp.float32),
            pltpu.VMEM((NPAD,), jnp.float32),
            pltpu.VMEM((128,), jnp.int32),
            pltpu.VMEM((128,), jnp.int32),
            pltpu.VMEM((KTOP + 2,), jnp.int32),
            pltpu.VMEM((KTOP + 2, 96), jnp.float32),
            pltpu.SemaphoreType.DMA,
        ],
    )


def _l1_body(dp_ref, x_ref, w_ref, batch_ref, o_hs, o_dinv,
             o_counts, o_starts):
    deg = dp_ref[0][:NNODES] + dp_ref[1][:NNODES] + 1.0
    dinv = lax.rsqrt(deg)
    o_dinv[...] = dinv
    o_hs[...] = jnp.dot(x_ref[...], w_ref[...],
                        preferred_element_type=jnp.float32) * dinv
    gids = lax.broadcasted_iota(jnp.int32, (128, NNODES), 0)
    onehot = (batch_ref[...] == gids).astype(jnp.float32)
    counts = jnp.sum(onehot, axis=1, keepdims=True)
    r = lax.broadcasted_iota(jnp.int32, (128, 128), 0)
    c = lax.broadcasted_iota(jnp.int32, (128, 128), 1)
    tril = (c < r).astype(jnp.float32)
    starts = jnp.dot(tril, counts, preferred_element_type=jnp.float32)
    o_counts[...] = counts.astype(jnp.int32)
    o_starts[...] = starts.astype(jnp.int32)


def _lmid_body(p_ref, hs_ref, dinv_ref, b_ref, wn_ref, o_a, o_hsn):
    dinv = dinv_ref[...]
    a = jnp.tanh(dinv * (p_ref[0][:NNODES] + p_ref[1][:NNODES]
                      + hs_ref[...]) + b_ref[...])
    o_a[...] = a
    o_hsn[...] = jnp.dot(a, wn_ref[...],
                         preferred_element_type=jnp.float32) * dinv


def _head_body(xkf_ref, w1e_ref, w1o_ref, b1_ref, w2_ref, b2_ref,
               wl1_ref, bl1_ref, wl2_ref, bl2_ref, o_ref):
    xkf = xkf_ref[...]
    ce = jnp.maximum(jnp.dot(xkf, w1e_ref[...],
                             preferred_element_type=jnp.float32)
                     + b1_ref[...], 0.0)
    co = jnp.maximum(jnp.dot(xkf, w1o_ref[...],
                             preferred_element_type=jnp.float32)
                     + b1_ref[...], 0.0)
    r1 = jnp.maximum(ce, co)
    r2 = jnp.maximum(jnp.dot(r1, w2_ref[...],
                             preferred_element_type=jnp.float32)
                     + b2_ref[...], 0.0)
    p = r2[:, 0:512]
    for k in range(1, 5):
        p = jnp.maximum(p, r2[:, k * 32:k * 32 + 512])
    h = jnp.maximum(jnp.dot(p, wl1_ref[...],
                            preferred_element_type=jnp.float32)
                    + bl1_ref[...], 0.0)
    logits = jnp.dot(h, wl2_ref[...],
                     preferred_element_type=jnp.float32) + bl2_ref[...]
    m = jnp.max(logits, axis=1, keepdims=True)
    shifted = logits - m
    o_ref[...] = shifted - jnp.log(jnp.sum(jnp.exp(shifted), axis=1,
                                           keepdims=True))


_SEL_E = np.zeros((96, 24, 2), np.float32)
_SEL_O = np.zeros((96, 24, 2), np.float32)
for _u in range(24):
    for _k in range(2):
        _SEL_E[4 * _u + _k, _u, _k] = 1.0
        if 4 * _u + 2 + _k < 96:
            _SEL_O[4 * _u + 2 + _k, _u, _k] = 1.0
_SEL2 = np.zeros((24, 20, 5), np.float32)
for _t in range(20):
    for _k in range(5):
        _SEL2[_t + _k, _t, _k] = 1.0


def _tc_call(body, out_shape):
    return pl.pallas_call(body, out_shape=out_shape)


def kernel(x, edge_index, batch, W1, b1, W2, b2, W3, b3, W4, b4,
           Wc1, bc1, Wc2, bc2, Wl1, bl1, Wl2, bl2):
    f32 = jnp.float32
    ndum = EPAD - NEDGES
    dum = np.arange(ndum, dtype=np.int32)
    epad = jnp.concatenate(
        [edge_index,
         jnp.asarray(np.stack([(dum * 7919) % NNODES,
                               NNODES + dum % (NPAD - NNODES)]))],
        axis=1)
    src = epad[0].reshape(NW * NCHUNK, CH)
    dst = epad[1].reshape(NW * NCHUNK, CH)

    zr1 = jnp.zeros((NPAD,), f32)
    zr32 = jnp.zeros((NPAD, 32), f32)
    zr16 = jnp.zeros((NPAD, 16), f32)

    degp = _make_deg()(dst, zr1)
    degp3 = degp.reshape(NC, NPAD, 1)

    hs1, dinv, counts2, starts2 = _tc_call(
        _l1_body,
        (jax.ShapeDtypeStruct((NNODES, 32), f32),
         jax.ShapeDtypeStruct((NNODES, 1), f32),
         jax.ShapeDtypeStruct((128, 1), jnp.int32),
         jax.ShapeDtypeStruct((128, 1), jnp.int32)),
    )(degp3, x, W1, batch.reshape(1, NNODES))

    mp32 = _make_mp(32)
    p1 = mp32(hs1, src, dst, zr32)
    a1, hs2 = _tc_call(
        _lmid_body,
        (jax.ShapeDtypeStruct((NNODES, 32), f32),
         jax.ShapeDtypeStruct((NNODES, 32), f32)),
    )(p1, hs1, dinv, b1.reshape(1, 32), W2)

    p2 = mp32(hs2, src, dst, zr32)
    a2, hs3 = _tc_call(
        _lmid_body,
        (jax.ShapeDtypeStruct((NNODES, 32), f32),
         jax.ShapeDtypeStruct((NNODES, 32), f32)),
    )(p2, hs2, dinv, b2.reshape(1, 32), W3)

    p3 = mp32(hs3, src, dst, zr32)
    W4p = jnp.pad(W4, ((0, 0), (0, 15)))
    a3, hs4 = _tc_call(
        _lmid_body,
        (jax.ShapeDtypeStruct((NNODES, 32), f32),
         jax.ShapeDtypeStruct((NNODES, 16), f32)),
    )(p3, hs3, dinv, b3.reshape(1, 32), W4p)

    hs4p = jnp.pad(hs4, ((0, NPAD - NNODES), (0, 0)))
    p4 = _make_mp(16)(hs4p, src, dst, zr16)

    xc = jnp.concatenate([a1, a2, a3], axis=1)
    dinvp = jnp.pad(dinv.reshape(NNODES), (0, NPAD - NNODES)).reshape(NS, ZSL)
    b4p = jnp.zeros((128,), f32).at[0].set(b4[0])
    starts1 = starts2.reshape(128)
    counts1 = counts2.reshape(128)

    xk = _make_topkz()(p4, hs4p, dinvp, b4p, xc, starts1, counts1)

    xkf = xk.reshape(NGRAPH, KTOP * 96)
    W1e = jnp.einsum('oik,huk->ihuo', Wc1,
                     jnp.asarray(_SEL_E)).reshape(KTOP * 96, 384)
    W1o = jnp.einsum('oik,huk->ihuo', Wc1,
                     jnp.asarray(_SEL_O)).reshape(KTOP * 96, 384)
    W2b = jnp.einsum('oik,utk->uito', Wc2,
                     jnp.asarray(_SEL2)).reshape(384, 640)
    Wl1p = Wl1.reshape(32, 16, 128).transpose(1, 0, 2).reshape(512, 128)

    out = _tc_call(
        _head_body, jax.ShapeDtypeStruct((NGRAPH, 10), f32),
    )(xkf, W1e, W1o, jnp.tile(bc1.reshape(1, 16), (1, 24)),
      W2b, jnp.tile(bc2.reshape(1, 32), (1, 20)),
      Wl1p, bl1.reshape(1, 128), Wl2, bl2.reshape(1, 10))
    return out


# speedup vs baseline: 1.4419x; 1.0259x over previous
import functools

import jax
import jax.numpy as jnp
import numpy as np
from jax import lax
from jax.experimental import pallas as pl
from jax.experimental.pallas import tpu as pltpu
from jax.experimental.pallas import tpu_sc as plsc

NC, NS, LANES = 2, 16, 16
NW = NC * NS
NNODES = 10000
NEDGES = 320000
CH = 128
NCHUNK = 79
EPAD = NW * NCHUNK * CH
NGRAPH = 100
KTOP = 30
NPAD = 10240
ZSL = NPAD // NS
NEGZ = -1e30
NEG = -3.0
BIG = 1 << 30


def _sc_mesh():
    return plsc.VectorSubcoreMesh(core_axis_name="c", subcore_axis_name="s",
                                  num_cores=NC, num_subcores=NS)


def _deg_body(ei_hbm, zr_hbm, out_hbm, dst_v, ones_v, acc_sh, dsem):
    cid = lax.axis_index("c")
    sid = lax.axis_index("s")
    wid = cid * NS + sid
    pltpu.sync_copy(ei_hbm.at[1, pl.ds(wid * NCHUNK, NCHUNK)], dst_v)
    for i in range(8):
        ones_v[pl.ds(i * LANES, LANES)] = jnp.full((LANES,), 1.0, jnp.float32)

    @pl.when(sid == 0)
    def _():
        pltpu.sync_copy(zr_hbm, acc_sh)
    plsc.subcore_barrier()

    def chunk(ch, carry):
        pltpu.async_copy(ones_v.at[pl.ds(0, CH)], acc_sh.at[dst_v.at[ch]],
                         dsem, add=True)
        return carry
    lax.fori_loop(0, NCHUNK, chunk, 0)

    def drain(ch, carry):
        pltpu.make_async_copy(ones_v.at[pl.ds(0, CH)],
                              acc_sh.at[dst_v.at[ch]], dsem).wait()
        return carry
    lax.fori_loop(0, NCHUNK, drain, 0)
    plsc.subcore_barrier()

    @pl.when(sid == 0)
    def _():
        pltpu.sync_copy(acc_sh, out_hbm.at[cid])


def _make_deg():
    return pl.kernel(
        _deg_body,
        out_type=jax.ShapeDtypeStruct((NC, NPAD), jnp.float32),
        mesh=_sc_mesh(),
        compiler_params=pltpu.CompilerParams(use_tc_tiling_on_sc=False, needs_layout_passes=False),
        scratch_types=[
            pltpu.VMEM((NCHUNK, CH), jnp.int32),
            pltpu.VMEM((128,), jnp.float32),
            pltpu.VMEM_SHARED((NPAD,), jnp.float32),
            pltpu.SemaphoreType.DMA,
        ],
    )


RING = 8
LOOKAHEAD = 4


def _mp_body(hs_hbm, ei_hbm, zr_hbm, out_hbm,
             src_v, dst_v, rows_v, acc_sh, gsems, ssems):
    cid = lax.axis_index("c")
    sid = lax.axis_index("s")
    wid = cid * NS + sid
    pltpu.sync_copy(ei_hbm.at[0, pl.ds(wid * NCHUNK, NCHUNK)], src_v)
    pltpu.sync_copy(ei_hbm.at[1, pl.ds(wid * NCHUNK, NCHUNK)], dst_v)

    @pl.when(sid == 0)
    def _():
        pltpu.sync_copy(zr_hbm, acc_sh)
    plsc.subcore_barrier()

    def gdesc(c, j):
        return pltpu.make_async_copy(hs_hbm.at[src_v.at[c]], rows_v.at[j],
                                     gsems.at[j])

    def sdesc(c, j):
        return pltpu.make_async_copy(rows_v.at[j], acc_sh.at[dst_v.at[c]],
                                     ssems.at[j])

    def step(i, carry):
        p0 = i * RING
        for j in range(RING):
            p = p0 + j
            ja = (j - LOOKAHEAD) % RING

            @pl.when((p >= RING) & (p < NCHUNK))
            def _():
                sdesc(p - RING, j).wait()

            @pl.when(p < NCHUNK)
            def _():
                pltpu.async_copy(hs_hbm.at[src_v.at[p]], rows_v.at[j],
                                 gsems.at[j])
            ca = p - LOOKAHEAD

            @pl.when((ca >= 0) & (ca < NCHUNK))
            def _():
                gdesc(ca, ja).wait()
                pltpu.async_copy(rows_v.at[ja], acc_sh.at[dst_v.at[ca]],
                                 ssems.at[ja], add=True)
        return carry
    nsteps = (NCHUNK + LOOKAHEAD + RING - 1) // RING
    lax.fori_loop(0, nsteps, step, 0)
    for j in range(RING):
        c = NCHUNK - RING + j
        sdesc(c, c % RING).wait()
    plsc.subcore_barrier()

    @pl.when(sid == 0)
    def _():
        pltpu.sync_copy(acc_sh, out_hbm.at[cid])


def _make_mp(feat, nnodes=NPAD):
    return pl.kernel(
        _mp_body,
        out_type=jax.ShapeDtypeStruct((NC, nnodes, feat), jnp.float32),
        mesh=_sc_mesh(),
        compiler_params=pltpu.CompilerParams(use_tc_tiling_on_sc=False, needs_layout_passes=False),
        scratch_types=[
            pltpu.VMEM((NCHUNK, CH), jnp.int32),
            pltpu.VMEM((NCHUNK, CH), jnp.int32),
            pltpu.VMEM((RING, CH, feat), jnp.float32),
            pltpu.VMEM_SHARED((nnodes, feat), jnp.float32),
            pltpu.SemaphoreType.DMA((RING,)),
            pltpu.SemaphoreType.DMA((RING,)),
        ],
    )


ZTAIL = NNODES - (NS - 1) * ZSL


def _topkz_body(p4_hbm, hs4_hbm, dinv_hbm, b4_hbm, x1_hbm, x2_hbm, x3_hbm,
                starts_hbm, counts_hbm, out_hbm,
                pa_v, pb_v, hb_v, dv_v, b4s_v, zl_v, z_sh, zv_v,
                starts_v, counts_v, idx_v, r1_v, r2_v, r3_v, sem):
    cid = lax.axis_index("c")
    sid = lax.axis_index("s")
    wid = cid * NS + sid
    iota = lax.iota(jnp.int32, LANES)
    lane0 = iota == 0
    base = sid * ZSL

    pltpu.sync_copy(p4_hbm.at[0, pl.ds(base, ZSL)], pa_v)
    pltpu.sync_copy(p4_hbm.at[1, pl.ds(base, ZSL)], pb_v)

    @pl.when(sid < NS - 1)
    def _():
        pltpu.sync_copy(hs4_hbm.at[pl.ds(base, ZSL)], hb_v)
        pltpu.sync_copy(dinv_hbm.at[pl.ds(base, ZSL)], dv_v)

    @pl.when(sid == NS - 1)
    def _():
        pltpu.sync_copy(hs4_hbm.at[pl.ds(base, ZTAIL)],
                        hb_v.at[pl.ds(0, ZTAIL)])
        pltpu.sync_copy(dinv_hbm.at[pl.ds(base, ZTAIL)],
                        dv_v.at[pl.ds(0, ZTAIL)])
    pltpu.sync_copy(b4_hbm, b4s_v)
    pltpu.sync_copy(starts_hbm, starts_v)
    pltpu.sync_copy(counts_hbm, counts_v)
    b4 = b4s_v[pl.ds(0, LANES)][0]
    zeros16 = jnp.zeros((LANES,), jnp.int32)

    def zchunk(i, carry):
        r16 = i * LANES + iota
        ga = plsc.load_gather(pa_v, [r16, zeros16])
        gb = plsc.load_gather(pb_v, [r16, zeros16])
        gh = plsc.load_gather(hb_v, [r16, zeros16])
        gd = plsc.load_gather(dv_v, [r16, zeros16])
        z16 = gd * (ga + gb + gh) + b4
        z16 = jnp.where(base + r16 < NNODES, z16, jnp.float32(1e30))
        zl_v[pl.ds(i * LANES, LANES)] = z16
        return carry
    lax.fori_loop(0, ZSL // LANES, zchunk, 0)
    pltpu.sync_copy(zl_v, z_sh.at[pl.ds(base, ZSL)])
    plsc.subcore_barrier()
    pltpu.sync_copy(z_sh, zv_v)

    def minchunk(i, m):
        return jnp.minimum(m, jnp.min(zv_v[pl.ds(i * LANES, LANES)]))
    zmin = lax.fori_loop(0, NNODES // LANES, minchunk, jnp.float32(1e30))
    zminv = jnp.full((LANES,), zmin, jnp.float32)
    fillv = -2.0 / (jnp.exp(2.0 * zminv) + 1.0)

    for rep in range(4):
        g = wid + rep * NW

        @pl.when(g < NGRAPH)
        def _process():
            start = starts_v[pl.ds(g, LANES)][0]
            cnt = counts_v[pl.ds(g, LANES)][0]
            end = start + cnt
            nsel = jnp.minimum(cnt, KTOP)
            c0 = start // LANES
            c1 = (end + LANES - 1) // LANES

            def select(k, carry):
                def scan_chunk(c, mbi):
                    m, bi = mbi
                    v = zv_v[pl.ds(c * LANES, LANES)]
                    p = c * LANES + iota
                    valid = (p >= start) & (p < end)
                    vv = jnp.where(valid, v, NEGZ)
                    cm = jnp.max(vv)
                    ci = jnp.min(jnp.where(vv == cm, p, BIG))
                    better = cm > m
                    return (jnp.where(better, cm, m),
                            jnp.where(better, ci, bi))
                m, bi = lax.fori_loop(c0, c1, scan_chunk,
                                      (jnp.float32(NEGZ), jnp.int32(BIG)))
                biv = jnp.full((LANES,), bi, jnp.int32)
                plsc.store_scatter(idx_v, [jnp.full((LANES,), k, jnp.int32)],
                                   biv, mask=lane0)
                plsc.store_scatter(zv_v, [biv],
                                   jnp.full((LANES,), NEGZ, jnp.float32),
                                   mask=lane0)
                return carry
            lax.fori_loop(0, nsel, select, 0)

            def pad(k, carry):
                plsc.store_scatter(idx_v, [jnp.full((LANES,), k, jnp.int32)],
                                   jnp.zeros((LANES,), jnp.int32), mask=lane0)
                return carry
            lax.fori_loop(nsel, KTOP + 2, pad, 0)

            pltpu.async_copy(x1_hbm.at[idx_v], r1_v, sem)
            pltpu.async_copy(x2_hbm.at[idx_v], r2_v, sem)
            pltpu.async_copy(x3_hbm.at[idx_v], r3_v, sem)
            pltpu.make_async_copy(x1_hbm.at[idx_v], r1_v, sem).wait()
            pltpu.make_async_copy(x2_hbm.at[idx_v], r2_v, sem).wait()
            pltpu.make_async_copy(x3_hbm.at[idx_v], r3_v, sem).wait()

            def fillrow(k, carry):
                for buf in (r1_v, r2_v, r3_v):
                    for j in range(2):
                        buf[k, pl.ds(j * LANES, LANES)] = fillv
                return carry
            lax.fori_loop(nsel, KTOP, fillrow, 0)
            pltpu.sync_copy(r1_v.at[pl.ds(0, KTOP)],
                            out_hbm.at[g, pl.ds(0, KTOP), pl.ds(0, 32)])
            pltpu.sync_copy(r2_v.at[pl.ds(0, KTOP)],
                            out_hbm.at[g, pl.ds(0, KTOP), pl.ds(32, 32)])
            pltpu.sync_copy(r3_v.at[pl.ds(0, KTOP)],
                            out_hbm.at[g, pl.ds(0, KTOP), pl.ds(64, 32)])


def _make_topkz():
    return pl.kernel(
        _topkz_body,
        out_type=jax.ShapeDtypeStruct((NGRAPH, KTOP, 96), jnp.float32),
        mesh=_sc_mesh(),
        compiler_params=pltpu.CompilerParams(use_tc_tiling_on_sc=False, needs_layout_passes=False),
        scratch_types=[
            pltpu.VMEM((ZSL, 16), jnp.float32),
            pltpu.VMEM((ZSL, 16), jnp.float32),
            pltpu.VMEM((ZSL, 16), jnp.float32),
            pltpu.VMEM((ZSL, 1), jnp.float32),
            pltpu.VMEM((128,), jnp.float32),
            pltpu.VMEM((ZSL,), jnp.float32),
            pltpu.VMEM_SHARED((NPAD,), jnp.float32),
            pltpu.VMEM((NPAD,), jnp.float32),
            pltpu.VMEM((128,), jnp.int32),
            pltpu.VMEM((128,), jnp.int32),
            pltpu.VMEM((KTOP + 2,), jnp.int32),
            pltpu.VMEM((KTOP + 2, 32), jnp.float32),
            pltpu.VMEM((KTOP + 2, 32), jnp.float32),
            pltpu.VMEM((KTOP + 2, 32), jnp.float32),
            pltpu.SemaphoreType.DMA,
        ],
    )


def _l1_body(dp_ref, x_ref, w_ref, batch_ref, o_hs, o_dinv,
             o_counts, o_starts):
    deg = dp_ref[0][:NNODES] + dp_ref[1][:NNODES] + 1.0
    dinv = lax.rsqrt(deg)
    o_dinv[...] = dinv
    o_hs[...] = jnp.dot(x_ref[...], w_ref[...],
                        preferred_element_type=jnp.float32) * dinv
    gids = lax.broadcasted_iota(jnp.int32, (128, NNODES), 0)
    onehot = (batch_ref[...] == gids).astype(jnp.float32)
    counts = jnp.sum(onehot, axis=1, keepdims=True)
    r = lax.broadcasted_iota(jnp.int32, (128, 128), 0)
    c = lax.broadcasted_iota(jnp.int32, (128, 128), 1)
    tril = (c < r).astype(jnp.float32)
    starts = jnp.dot(tril, counts, preferred_element_type=jnp.float32)
    o_counts[...] = counts.astype(jnp.int32)
    o_starts[...] = starts.astype(jnp.int32)


def _lmid_body(p_ref, hs_ref, dinv_ref, b_ref, wn_ref, o_a, o_hsn):
    dinv = dinv_ref[...]
    a = jnp.tanh(dinv * (p_ref[0][:NNODES] + p_ref[1][:NNODES]
                      + hs_ref[...]) + b_ref[...])
    o_a[...] = a
    o_hsn[...] = jnp.dot(a, wn_ref[...],
                         preferred_element_type=jnp.float32) * dinv


def _head_body(xkf_ref, w1e_ref, w1o_ref, b1_ref, w2_ref, b2_ref,
               wl1_ref, bl1_ref, wl2_ref, bl2_ref, o_ref):
    xkf = xkf_ref[...]
    ce = jnp.maximum(jnp.dot(xkf, w1e_ref[...],
                             preferred_element_type=jnp.float32)
                     + b1_ref[...], 0.0)
    co = jnp.maximum(jnp.dot(xkf, w1o_ref[...],
                             preferred_element_type=jnp.float32)
                     + b1_ref[...], 0.0)
    r1 = jnp.maximum(ce, co)
    r2 = jnp.maximum(jnp.dot(r1, w2_ref[...],
                             preferred_element_type=jnp.float32)
                     + b2_ref[...], 0.0)
    p = r2[:, 0:512]
    for k in range(1, 5):
        p = jnp.maximum(p, r2[:, k * 32:k * 32 + 512])
    h = jnp.maximum(jnp.dot(p, wl1_ref[...],
                            preferred_element_type=jnp.float32)
                    + bl1_ref[...], 0.0)
    logits = jnp.dot(h, wl2_ref[...],
                     preferred_element_type=jnp.float32) + bl2_ref[...]
    m = jnp.max(logits, axis=1, keepdims=True)
    shifted = logits - m
    o_ref[...] = shifted - jnp.log(jnp.sum(jnp.exp(shifted), axis=1,
                                           keepdims=True))


_SEL_E = np.zeros((96, 24, 2), np.float32)
_SEL_O = np.zeros((96, 24, 2), np.float32)
for _u in range(24):
    for _k in range(2):
        _SEL_E[4 * _u + _k, _u, _k] = 1.0
        if 4 * _u + 2 + _k < 96:
            _SEL_O[4 * _u + 2 + _k, _u, _k] = 1.0
_SEL2 = np.zeros((24, 20, 5), np.float32)
for _t in range(20):
    for _k in range(5):
        _SEL2[_t + _k, _t, _k] = 1.0


def _tc_call(body, out_shape):
    return pl.pallas_call(body, out_shape=out_shape)


def kernel(x, edge_index, batch, W1, b1, W2, b2, W3, b3, W4, b4,
           Wc1, bc1, Wc2, bc2, Wl1, bl1, Wl2, bl2):
    f32 = jnp.float32
    ndum = EPAD - NEDGES
    dum = np.arange(ndum, dtype=np.int32)
    epad = jnp.concatenate(
        [edge_index,
         jnp.asarray(np.stack([(dum * 7919) % NNODES,
                               NNODES + dum % (NPAD - NNODES)]))],
        axis=1)
    ei3 = epad.reshape(2, NW * NCHUNK, CH)

    zr1 = jnp.zeros((NPAD,), f32)
    zr32 = jnp.zeros((NPAD, 32), f32)
    zr16 = jnp.zeros((NPAD, 16), f32)

    degp = _make_deg()(ei3, zr1)
    degp3 = degp.reshape(NC, NPAD, 1)

    hs1, dinv, counts2, starts2 = _tc_call(
        _l1_body,
        (jax.ShapeDtypeStruct((NNODES, 32), f32),
         jax.ShapeDtypeStruct((NNODES, 1), f32),
         jax.ShapeDtypeStruct((128, 1), jnp.int32),
         jax.ShapeDtypeStruct((128, 1), jnp.int32)),
    )(degp3, x, W1, batch.reshape(1, NNODES))

    mp32 = _make_mp(32)
    p1 = mp32(hs1, ei3, zr32)
    a1, hs2 = _tc_call(
        _lmid_body,
        (jax.ShapeDtypeStruct((NNODES, 32), f32),
         jax.ShapeDtypeStruct((NNODES, 32), f32)),
    )(p1, hs1, dinv, b1.reshape(1, 32), W2)

    p2 = mp32(hs2, ei3, zr32)
    a2, hs3 = _tc_call(
        _lmid_body,
        (jax.ShapeDtypeStruct((NNODES, 32), f32),
         jax.ShapeDtypeStruct((NNODES, 32), f32)),
    )(p2, hs2, dinv, b2.reshape(1, 32), W3)

    p3 = mp32(hs3, ei3, zr32)
    W4p = jnp.pad(W4, ((0, 0), (0, 15)))
    a3, hs4 = _tc_call(
        _lmid_body,
        (jax.ShapeDtypeStruct((NNODES, 32), f32),
         jax.ShapeDtypeStruct((NNODES, 16), f32)),
    )(p3, hs3, dinv, b3.reshape(1, 32), W4p)

    p4 = _make_mp(16)(hs4, ei3, zr16)

    b4p = jnp.zeros((128,), f32).at[0].set(b4[0])
    starts1 = starts2.reshape(128)
    counts1 = counts2.reshape(128)

    xk = _make_topkz()(p4, hs4, dinv, b4p, a1, a2, a3, starts1, counts1)

    xkf = xk.reshape(NGRAPH, KTOP * 96)
    W1e = jnp.einsum('oik,huk->ihuo', Wc1,
                     jnp.asarray(_SEL_E)).reshape(KTOP * 96, 384)
    W1o = jnp.einsum('oik,huk->ihuo', Wc1,
                     jnp.asarray(_SEL_O)).reshape(KTOP * 96, 384)
    W2b = jnp.einsum('oik,utk->uito', Wc2,
                     jnp.asarray(_SEL2)).reshape(384, 640)
    Wl1p = Wl1.reshape(32, 16, 128).transpose(1, 0, 2).reshape(512, 128)

    out = _tc_call(
        _head_body, jax.ShapeDtypeStruct((NGRAPH, 10), f32),
    )(xkf, W1e, W1o, jnp.tile(bc1.reshape(1, 16), (1, 24)),
      W2b, jnp.tile(bc2.reshape(1, 32), (1, 20)),
      Wl1p, bl1.reshape(1, 128), Wl2, bl2.reshape(1, 10))
    return out


# speedup vs baseline: 1.4505x; 1.0059x over previous
import functools

import jax
import jax.numpy as jnp
import numpy as np
from jax import lax
from jax.experimental import pallas as pl
from jax.experimental.pallas import tpu as pltpu
from jax.experimental.pallas import tpu_sc as plsc

NC, NS, LANES = 2, 16, 16
NW = NC * NS
NNODES = 10000
NEDGES = 320000
CH = 128
NCHUNK = 79
EPAD = NW * NCHUNK * CH
NGRAPH = 100
KTOP = 30
NPAD = 10240
ZSL = NPAD // NS
NEGZ = -1e30
NEG = -3.0
BIG = 1 << 30


def _sc_mesh():
    return plsc.VectorSubcoreMesh(core_axis_name="c", subcore_axis_name="s",
                                  num_cores=NC, num_subcores=NS)


def _deg_body(ei_hbm, zr_hbm, out_hbm, dst_v, ones_v, acc_sh, dsem):
    cid = lax.axis_index("c")
    sid = lax.axis_index("s")
    wid = cid * NS + sid
    pltpu.sync_copy(ei_hbm.at[1, pl.ds(wid * NCHUNK, NCHUNK)], dst_v)
    for i in range(8):
        ones_v[pl.ds(i * LANES, LANES)] = jnp.full((LANES,), 1.0, jnp.float32)

    @pl.when(sid == 0)
    def _():
        pltpu.sync_copy(zr_hbm, acc_sh)
    plsc.subcore_barrier()

    def chunk(ch, carry):
        pltpu.async_copy(ones_v.at[pl.ds(0, CH)], acc_sh.at[dst_v.at[ch]],
                         dsem, add=True)
        return carry
    lax.fori_loop(0, NCHUNK, chunk, 0)

    def drain(ch, carry):
        pltpu.make_async_copy(ones_v.at[pl.ds(0, CH)],
                              acc_sh.at[dst_v.at[ch]], dsem).wait()
        return carry
    lax.fori_loop(0, NCHUNK, drain, 0)
    plsc.subcore_barrier()

    @pl.when(sid == 0)
    def _():
        pltpu.sync_copy(acc_sh, out_hbm.at[cid])


def _make_deg():
    return pl.kernel(
        _deg_body,
        out_type=jax.ShapeDtypeStruct((NC, NPAD), jnp.float32),
        mesh=_sc_mesh(),
        compiler_params=pltpu.CompilerParams(use_tc_tiling_on_sc=False, needs_layout_passes=False),
        scratch_types=[
            pltpu.VMEM((NCHUNK, CH), jnp.int32),
            pltpu.VMEM((128,), jnp.float32),
            pltpu.VMEM_SHARED((NPAD,), jnp.float32),
            pltpu.SemaphoreType.DMA,
        ],
    )


RING = 12
LOOKAHEAD = 6


def _mp_body(hs_hbm, ei_hbm, zr_hbm, out_hbm,
             src_v, dst_v, rows_v, acc_sh, gsems, ssems):
    cid = lax.axis_index("c")
    sid = lax.axis_index("s")
    wid = cid * NS + sid
    pltpu.sync_copy(ei_hbm.at[0, pl.ds(wid * NCHUNK, NCHUNK)], src_v)
    pltpu.sync_copy(ei_hbm.at[1, pl.ds(wid * NCHUNK, NCHUNK)], dst_v)

    @pl.when(sid == 0)
    def _():
        pltpu.sync_copy(zr_hbm, acc_sh)
    plsc.subcore_barrier()

    def gdesc(c, j):
        return pltpu.make_async_copy(hs_hbm.at[src_v.at[c]], rows_v.at[j],
                                     gsems.at[j])

    def sdesc(c, j):
        return pltpu.make_async_copy(rows_v.at[j], acc_sh.at[dst_v.at[c]],
                                     ssems.at[j])

    def step(i, carry):
        p0 = i * RING
        for j in range(RING):
            p = p0 + j
            ja = (j - LOOKAHEAD) % RING

            @pl.when((p >= RING) & (p < NCHUNK))
            def _():
                sdesc(p - RING, j).wait()

            @pl.when(p < NCHUNK)
            def _():
                pltpu.async_copy(hs_hbm.at[src_v.at[p]], rows_v.at[j],
                                 gsems.at[j])
            ca = p - LOOKAHEAD

            @pl.when((ca >= 0) & (ca < NCHUNK))
            def _():
                gdesc(ca, ja).wait()
                pltpu.async_copy(rows_v.at[ja], acc_sh.at[dst_v.at[ca]],
                                 ssems.at[ja], add=True)
        return carry
    nsteps = (NCHUNK + LOOKAHEAD + RING - 1) // RING
    lax.fori_loop(0, nsteps, step, 0)
    for j in range(RING):
        c = NCHUNK - RING + j
        sdesc(c, c % RING).wait()
    plsc.subcore_barrier()

    @pl.when(sid == 0)
    def _():
        pltpu.sync_copy(acc_sh, out_hbm.at[cid])


def _make_mp(feat, nnodes=NPAD):
    return pl.kernel(
        _mp_body,
        out_type=jax.ShapeDtypeStruct((NC, nnodes, feat), jnp.float32),
        mesh=_sc_mesh(),
        compiler_params=pltpu.CompilerParams(use_tc_tiling_on_sc=False, needs_layout_passes=False),
        scratch_types=[
            pltpu.VMEM((NCHUNK, CH), jnp.int32),
            pltpu.VMEM((NCHUNK, CH), jnp.int32),
            pltpu.VMEM((RING, CH, feat), jnp.float32),
            pltpu.VMEM_SHARED((nnodes, feat), jnp.float32),
            pltpu.SemaphoreType.DMA((RING,)),
            pltpu.SemaphoreType.DMA((RING,)),
        ],
    )


ZTAIL = NNODES - (NS - 1) * ZSL


def _topkz_body(p4_hbm, hs4_hbm, dinv_hbm, b4_hbm, x1_hbm, x2_hbm, x3_hbm,
                starts_hbm, counts_hbm, out_hbm,
                pa_v, pb_v, hb_v, dv_v, b4s_v, zl_v, z_sh, zv_v,
                starts_v, counts_v, idx_v, r1_v, r2_v, r3_v, sem):
    cid = lax.axis_index("c")
    sid = lax.axis_index("s")
    wid = cid * NS + sid
    iota = lax.iota(jnp.int32, LANES)
    lane0 = iota == 0
    base = sid * ZSL

    pltpu.sync_copy(p4_hbm.at[0, pl.ds(base, ZSL)], pa_v)
    pltpu.sync_copy(p4_hbm.at[1, pl.ds(base, ZSL)], pb_v)

    @pl.when(sid < NS - 1)
    def _():
        pltpu.sync_copy(hs4_hbm.at[pl.ds(base, ZSL)], hb_v)
        pltpu.sync_copy(dinv_hbm.at[pl.ds(base, ZSL)], dv_v)

    @pl.when(sid == NS - 1)
    def _():
        pltpu.sync_copy(hs4_hbm.at[pl.ds(base, ZTAIL)],
                        hb_v.at[pl.ds(0, ZTAIL)])
        pltpu.sync_copy(dinv_hbm.at[pl.ds(base, ZTAIL)],
                        dv_v.at[pl.ds(0, ZTAIL)])
    pltpu.sync_copy(b4_hbm, b4s_v)
    pltpu.sync_copy(starts_hbm, starts_v)
    pltpu.sync_copy(counts_hbm, counts_v)
    b4 = b4s_v[pl.ds(0, LANES)][0]
    zeros16 = jnp.zeros((LANES,), jnp.int32)

    def zchunk(i, carry):
        r16 = i * LANES + iota
        ga = plsc.load_gather(pa_v, [r16, zeros16])
        gb = plsc.load_gather(pb_v, [r16, zeros16])
        gh = plsc.load_gather(hb_v, [r16, zeros16])
        gd = plsc.load_gather(dv_v, [r16, zeros16])
        z16 = gd * (ga + gb + gh) + b4
        z16 = jnp.where(base + r16 < NNODES, z16, jnp.float32(1e30))
        zl_v[pl.ds(i * LANES, LANES)] = z16
        return carry
    lax.fori_loop(0, ZSL // LANES, zchunk, 0)
    pltpu.sync_copy(zl_v, z_sh.at[pl.ds(base, ZSL)])
    plsc.subcore_barrier()
    pltpu.sync_copy(z_sh, zv_v)

    def minchunk(i, m):
        return jnp.minimum(m, jnp.min(zv_v[pl.ds(i * LANES, LANES)]))
    zmin = lax.fori_loop(0, NNODES // LANES, minchunk, jnp.float32(1e30))
    zminv = jnp.full((LANES,), zmin, jnp.float32)
    fillv = -2.0 / (jnp.exp(2.0 * zminv) + 1.0)

    for rep in range(4):
        g = wid + rep * NW

        @pl.when(g < NGRAPH)
        def _process():
            start = starts_v[pl.ds(g, LANES)][0]
            cnt = counts_v[pl.ds(g, LANES)][0]
            end = start + cnt
            nsel = jnp.minimum(cnt, KTOP)
            c0 = start // LANES
            c1 = (end + LANES - 1) // LANES

            def select(k, carry):
                def scan_chunk(c, mbi):
                    m, bi = mbi
                    v = zv_v[pl.ds(c * LANES, LANES)]
                    p = c * LANES + iota
                    valid = (p >= start) & (p < end)
                    vv = jnp.where(valid, v, NEGZ)
                    cm = jnp.max(vv)
                    ci = jnp.min(jnp.where(vv == cm, p, BIG))
                    better = cm > m
                    return (jnp.where(better, cm, m),
                            jnp.where(better, ci, bi))
                m, bi = lax.fori_loop(c0, c1, scan_chunk,
                                      (jnp.float32(NEGZ), jnp.int32(BIG)))
                biv = jnp.full((LANES,), bi, jnp.int32)
                plsc.store_scatter(idx_v, [jnp.full((LANES,), k, jnp.int32)],
                                   biv, mask=lane0)
                plsc.store_scatter(zv_v, [biv],
                                   jnp.full((LANES,), NEGZ, jnp.float32),
                                   mask=lane0)
                return carry
            lax.fori_loop(0, nsel, select, 0)

            def pad(k, carry):
                plsc.store_scatter(idx_v, [jnp.full((LANES,), k, jnp.int32)],
                                   jnp.zeros((LANES,), jnp.int32), mask=lane0)
                return carry
            lax.fori_loop(nsel, KTOP + 2, pad, 0)

            pltpu.async_copy(x1_hbm.at[idx_v], r1_v, sem)
            pltpu.async_copy(x2_hbm.at[idx_v], r2_v, sem)
            pltpu.async_copy(x3_hbm.at[idx_v], r3_v, sem)
            pltpu.make_async_copy(x1_hbm.at[idx_v], r1_v, sem).wait()
            pltpu.make_async_copy(x2_hbm.at[idx_v], r2_v, sem).wait()
            pltpu.make_async_copy(x3_hbm.at[idx_v], r3_v, sem).wait()

            def fillrow(k, carry):
                for buf in (r1_v, r2_v, r3_v):
                    for j in range(2):
                        buf[k, pl.ds(j * LANES, LANES)] = fillv
                return carry
            lax.fori_loop(nsel, KTOP, fillrow, 0)
            pltpu.sync_copy(r1_v.at[pl.ds(0, KTOP)],
                            out_hbm.at[g, pl.ds(0, KTOP), pl.ds(0, 32)])
            pltpu.sync_copy(r2_v.at[pl.ds(0, KTOP)],
                            out_hbm.at[g, pl.ds(0, KTOP), pl.ds(32, 32)])
            pltpu.sync_copy(r3_v.at[pl.ds(0, KTOP)],
                            out_hbm.at[g, pl.ds(0, KTOP), pl.ds(64, 32)])


def _make_topkz():
    return pl.kernel(
        _topkz_body,
        out_type=jax.ShapeDtypeStruct((NGRAPH, KTOP, 96), jnp.float32),
        mesh=_sc_mesh(),
        compiler_params=pltpu.CompilerParams(use_tc_tiling_on_sc=False, needs_layout_passes=False),
        scratch_types=[
            pltpu.VMEM((ZSL, 16), jnp.float32),
            pltpu.VMEM((ZSL, 16), jnp.float32),
            pltpu.VMEM((ZSL, 16), jnp.float32),
            pltpu.VMEM((ZSL, 1), jnp.float32),
            pltpu.VMEM((128,), jnp.float32),
            pltpu.VMEM((ZSL,), jnp.float32),
            pltpu.VMEM_SHARED((NPAD,), jnp.float32),
            pltpu.VMEM((NPAD,), jnp.float32),
            pltpu.VMEM((128,), jnp.int32),
            pltpu.VMEM((128,), jnp.int32),
            pltpu.VMEM((KTOP + 2,), jnp.int32),
            pltpu.VMEM((KTOP + 2, 32), jnp.float32),
            pltpu.VMEM((KTOP + 2, 32), jnp.float32),
            pltpu.VMEM((KTOP + 2, 32), jnp.float32),
            pltpu.SemaphoreType.DMA,
        ],
    )


def _l1_body(dp_ref, x_ref, w_ref, batch_ref, o_hs, o_dinv,
             o_counts, o_starts):
    deg = dp_ref[0][:NNODES] + dp_ref[1][:NNODES] + 1.0
    dinv = lax.rsqrt(deg)
    o_dinv[...] = dinv
    o_hs[...] = jnp.dot(x_ref[...], w_ref[...],
                        preferred_element_type=jnp.float32) * dinv
    gids = lax.broadcasted_iota(jnp.int32, (128, NNODES), 0)
    onehot = (batch_ref[...] == gids).astype(jnp.float32)
    counts = jnp.sum(onehot, axis=1, keepdims=True)
    r = lax.broadcasted_iota(jnp.int32, (128, 128), 0)
    c = lax.broadcasted_iota(jnp.int32, (128, 128), 1)
    tril = (c < r).astype(jnp.float32)
    starts = jnp.dot(tril, counts, preferred_element_type=jnp.float32)
    o_counts[...] = counts.astype(jnp.int32)
    o_starts[...] = starts.astype(jnp.int32)


def _lmid_body(p_ref, hs_ref, dinv_ref, b_ref, wn_ref, o_a, o_hsn):
    dinv = dinv_ref[...]
    a = jnp.tanh(dinv * (p_ref[0][:NNODES] + p_ref[1][:NNODES]
                      + hs_ref[...]) + b_ref[...])
    o_a[...] = a
    o_hsn[...] = jnp.dot(a, wn_ref[...],
                         preferred_element_type=jnp.float32) * dinv


def _head_body(xkf_ref, w1e_ref, w1o_ref, b1_ref, w2_ref, b2_ref,
               wl1_ref, bl1_ref, wl2_ref, bl2_ref, o_ref):
    xkf = xkf_ref[...]
    ce = jnp.maximum(jnp.dot(xkf, w1e_ref[...],
                             preferred_element_type=jnp.float32)
                     + b1_ref[...], 0.0)
    co = jnp.maximum(jnp.dot(xkf, w1o_ref[...],
                             preferred_element_type=jnp.float32)
                     + b1_ref[...], 0.0)
    r1 = jnp.maximum(ce, co)
    r2 = jnp.maximum(jnp.dot(r1, w2_ref[...],
                             preferred_element_type=jnp.float32)
                     + b2_ref[...], 0.0)
    p = r2[:, 0:512]
    for k in range(1, 5):
        p = jnp.maximum(p, r2[:, k * 32:k * 32 + 512])
    h = jnp.maximum(jnp.dot(p, wl1_ref[...],
                            preferred_element_type=jnp.float32)
                    + bl1_ref[...], 0.0)
    logits = jnp.dot(h, wl2_ref[...],
                     preferred_element_type=jnp.float32) + bl2_ref[...]
    m = jnp.max(logits, axis=1, keepdims=True)
    shifted = logits - m
    o_ref[...] = shifted - jnp.log(jnp.sum(jnp.exp(shifted), axis=1,
                                           keepdims=True))


_SEL_E = np.zeros((96, 24, 2), np.float32)
_SEL_O = np.zeros((96, 24, 2), np.float32)
for _u in range(24):
    for _k in range(2):
        _SEL_E[4 * _u + _k, _u, _k] = 1.0
        if 4 * _u + 2 + _k < 96:
            _SEL_O[4 * _u + 2 + _k, _u, _k] = 1.0
_SEL2 = np.zeros((24, 20, 5), np.float32)
for _t in range(20):
    for _k in range(5):
        _SEL2[_t + _k, _t, _k] = 1.0


def _tc_call(body, out_shape):
    return pl.pallas_call(body, out_shape=out_shape)


def kernel(x, edge_index, batch, W1, b1, W2, b2, W3, b3, W4, b4,
           Wc1, bc1, Wc2, bc2, Wl1, bl1, Wl2, bl2):
    f32 = jnp.float32
    ndum = EPAD - NEDGES
    dum = np.arange(ndum, dtype=np.int32)
    epad = jnp.concatenate(
        [edge_index,
         jnp.asarray(np.stack([(dum * 7919) % NNODES,
                               NNODES + dum % (NPAD - NNODES)]))],
        axis=1)
    ei3 = epad.reshape(2, NW * NCHUNK, CH)

    zr1 = jnp.zeros((NPAD,), f32)
    zr32 = jnp.zeros((NPAD, 32), f32)
    zr16 = jnp.zeros((NPAD, 16), f32)

    degp = _make_deg()(ei3, zr1)
    degp3 = degp.reshape(NC, NPAD, 1)

    hs1, dinv, counts2, starts2 = _tc_call(
        _l1_body,
        (jax.ShapeDtypeStruct((NNODES, 32), f32),
         jax.ShapeDtypeStruct((NNODES, 1), f32),
         jax.ShapeDtypeStruct((128, 1), jnp.int32),
         jax.ShapeDtypeStruct((128, 1), jnp.int32)),
    )(degp3, x, W1, batch.reshape(1, NNODES))

    mp32 = _make_mp(32)
    p1 = mp32(hs1, ei3, zr32)
    a1, hs2 = _tc_call(
        _lmid_body,
        (jax.ShapeDtypeStruct((NNODES, 32), f32),
         jax.ShapeDtypeStruct((NNODES, 32), f32)),
    )(p1, hs1, dinv, b1.reshape(1, 32), W2)

    p2 = mp32(hs2, ei3, zr32)
    a2, hs3 = _tc_call(
        _lmid_body,
        (jax.ShapeDtypeStruct((NNODES, 32), f32),
         jax.ShapeDtypeStruct((NNODES, 32), f32)),
    )(p2, hs2, dinv, b2.reshape(1, 32), W3)

    p3 = mp32(hs3, ei3, zr32)
    W4p = jnp.pad(W4, ((0, 0), (0, 15)))
    a3, hs4 = _tc_call(
        _lmid_body,
        (jax.ShapeDtypeStruct((NNODES, 32), f32),
         jax.ShapeDtypeStruct((NNODES, 16), f32)),
    )(p3, hs3, dinv, b3.reshape(1, 32), W4p)

    p4 = _make_mp(16)(hs4, ei3, zr16)

    b4p = jnp.zeros((128,), f32).at[0].set(b4[0])
    starts1 = starts2.reshape(128)
    counts1 = counts2.reshape(128)

    xk = _make_topkz()(p4, hs4, dinv, b4p, a1, a2, a3, starts1, counts1)

    xkf = xk.reshape(NGRAPH, KTOP * 96)
    W1e = jnp.einsum('oik,huk->ihuo', Wc1,
                     jnp.asarray(_SEL_E)).reshape(KTOP * 96, 384)
    W1o = jnp.einsum('oik,huk->ihuo', Wc1,
                     jnp.asarray(_SEL_O)).reshape(KTOP * 96, 384)
    W2b = jnp.einsum('oik,utk->uito', Wc2,
                     jnp.asarray(_SEL2)).reshape(384, 640)
    Wl1p = Wl1.reshape(32, 16, 128).transpose(1, 0, 2).reshape(512, 128)

    out = _tc_call(
        _head_body, jax.ShapeDtypeStruct((NGRAPH, 10), f32),
    )(xkf, W1e, W1o, jnp.tile(bc1.reshape(1, 16), (1, 24)),
      W2b, jnp.tile(bc2.reshape(1, 32), (1, 20)),
      Wl1p, bl1.reshape(1, 128), Wl2, bl2.reshape(1, 10))
    return out
